# Initial kernel scaffold; baseline (speedup 1.0000x reference)
import jax
import jax.numpy as jnp
from jax.experimental import pallas as pl


def kernel(xyz):
    raise NotImplementedError("write your pallas kernel here")


# speedup vs baseline: 2800.5263x; 2800.5263x over previous
import jax
import jax.numpy as jnp
from jax.experimental import pallas as pl
from jax.experimental.pallas import tpu as pltpu

_B = 8
_N = 32768
_NPOINT = 1024


def _fps_kernel(x_ref, y_ref, z_ref, ox_ref, oy_ref, oz_ref, dist_ref):
    dist_ref[...] = jnp.zeros((_B, _N), jnp.float32)
    gidx = jax.lax.broadcasted_iota(jnp.int32, (_B, _N), 1)
    col = jax.lax.broadcasted_iota(jnp.int32, (_B, _NPOINT), 1)

    ox_ref[...] = jnp.broadcast_to(x_ref[:, 0:1], (_B, _NPOINT))
    oy_ref[...] = jnp.broadcast_to(y_ref[:, 0:1], (_B, _NPOINT))
    oz_ref[...] = jnp.broadcast_to(z_ref[:, 0:1], (_B, _NPOINT))

    def cond(carry):
        i, _, _, _, _, done = carry
        return jnp.logical_and(i < _NPOINT, jnp.logical_not(done))

    def body(carry):
        i, lx, ly, lz, prev_idx, _ = carry
        dx = x_ref[...] - lx
        dy = y_ref[...] - ly
        dz = z_ref[...] - lz
        d = jnp.sqrt((dx * dx + dy * dy) + dz * dz)
        nd = jnp.maximum(dist_ref[...], d)
        dist_ref[...] = nd
        m = jnp.max(nd, axis=1, keepdims=True)
        cand = jnp.where(nd == m, gidx, _N)
        idx = jnp.min(cand, axis=1, keepdims=True)
        sel = gidx == idx
        ninf = jnp.float32(-jnp.inf)
        nlx = jnp.max(jnp.where(sel, x_ref[...], ninf), axis=1, keepdims=True)
        nly = jnp.max(jnp.where(sel, y_ref[...], ninf), axis=1, keepdims=True)
        nlz = jnp.max(jnp.where(sel, z_ref[...], ninf), axis=1, keepdims=True)
        here = col == i
        ox_ref[...] = jnp.where(here, nlx, ox_ref[...])
        oy_ref[...] = jnp.where(here, nly, oy_ref[...])
        oz_ref[...] = jnp.where(here, nlz, oz_ref[...])
        done = jnp.all(idx == prev_idx)
        return i + 1, nlx, nly, nlz, idx, done

    zero_col = jnp.zeros((_B, 1), jnp.float32)
    init = (
        jnp.int32(1),
        x_ref[:, 0:1] + zero_col,
        y_ref[:, 0:1] + zero_col,
        z_ref[:, 0:1] + zero_col,
        jnp.zeros((_B, 1), jnp.int32),
        jnp.bool_(False),
    )
    i_fin, lx, ly, lz, _, _ = jax.lax.while_loop(cond, body, init)

    fill = col >= i_fin
    ox_ref[...] = jnp.where(fill, lx, ox_ref[...])
    oy_ref[...] = jnp.where(fill, ly, oy_ref[...])
    oz_ref[...] = jnp.where(fill, lz, oz_ref[...])


def kernel(xyz):
    x = xyz[:, :, 0]
    y = xyz[:, :, 1]
    z = xyz[:, :, 2]
    out_shape = jax.ShapeDtypeStruct((_B, _NPOINT), jnp.float32)
    ox, oy, oz = pl.pallas_call(
        _fps_kernel,
        out_shape=(out_shape, out_shape, out_shape),
        scratch_shapes=[pltpu.VMEM((_B, _N), jnp.float32)],
    )(x, y, z)
    return jnp.stack([ox, oy, oz], axis=-1)
